# Initial kernel scaffold; baseline (speedup 1.0000x reference)
import jax
import jax.numpy as jnp
from jax.experimental import pallas as pl
from jax.experimental.pallas import tpu as pltpu

N = 10000
D_IN = 128
HID = 64
OUT = 32
EMB = 128
HEADS = 2


def _h0_body(x_ref, n2v_ref, Wn_ref, bn_ref, Win_ref, bin_ref, Wg_ref, bg_ref,
             o_ref):
    x = x_ref[...]
    emb_p = n2v_ref[...] @ Wn_ref[...].T + bn_ref[...]
    Win = Win_ref[...]
    raw = x @ Win[:, :D_IN].T + bin_ref[...]
    xp = raw + emb_p @ Win[:, D_IN:].T
    Wg = Wg_ref[...]
    g = jax.nn.sigmoid(x @ Wg[:, :D_IN].T + emb_p @ Wg[:, D_IN:].T + bg_ref[...])
    o_ref[...] = g * xp + (1.0 - g) * raw


def _h0(x, n2v_table, W_n2vp, b_n2vp, W_in, b_in, W_gate, b_gate):
    return pl.pallas_call(
        _h0_body,
        out_shape=jax.ShapeDtypeStruct((N, HID), jnp.float32),
    )(x, n2v_table, W_n2vp, b_n2vp.reshape(1, HID), W_in,
      b_in.reshape(1, HID), W_gate, b_gate.reshape(1, HID))


def _sage(h, src, dst, Wl, bl, Wr, n, deg):
    s = jax.ops.segment_sum(h[src], dst, num_segments=n)
    mean = s / jnp.maximum(deg, 1.0)[:, None]
    return mean @ Wl.T + bl + h @ Wr.T


def _gat(h, src, dst, W, a_s, a_d, b, n):
    xw = (h @ W.T).reshape(n, HEADS, HID)
    al_s = (xw * a_s[None, :, :]).sum(-1)
    al_d = (xw * a_d[None, :, :]).sum(-1)
    e = jax.nn.leaky_relu(al_s[src] + al_d[dst], 0.2)
    m = jnp.max(al_s) + jnp.max(al_d)
    ex = jnp.exp(e - m)
    denom = jax.ops.segment_sum(ex, dst, num_segments=n)
    num = jax.ops.segment_sum(ex[:, :, None] * xw[src], dst, num_segments=n)
    out = num / jnp.where(denom > 0, denom, 1.0)[:, :, None]
    return out.mean(axis=1) + b


def kernel(x, edge_index, n2v_table, W_n2vp, b_n2vp, W_in, b_in, W_gate,
           b_gate, Wl1, bl1, Wr1, Wl2, bl2, Wr2, W_gat, att_src, att_dst,
           b_gat, Wl3, bl3, Wr3):
    n = x.shape[0]
    src, dst = edge_index[0], edge_index[1]
    h0 = _h0(x, n2v_table, W_n2vp, b_n2vp, W_in, b_in, W_gate, b_gate)
    deg = jax.ops.segment_sum(jnp.ones((src.shape[0],), jnp.float32), dst,
                              num_segments=n)
    h1 = jax.nn.relu(_sage(h0, src, dst, Wl1, bl1, Wr1, n, deg))
    h2 = jax.nn.relu(_sage(h1, src, dst, Wl2, bl2, Wr2, n, deg))
    h3 = jax.nn.relu(_gat(h2, src, dst, W_gat, att_src, att_dst, b_gat, n))
    out = _sage(h3, src, dst, Wl3, bl3, Wr3, n, deg)
    return out


# speedup vs baseline: 1.0046x; 1.0046x over previous
import jax
import jax.numpy as jnp
from jax.experimental import pallas as pl
from jax.experimental.pallas import tpu as pltpu

N = 10000
D_IN = 128
HID = 64
OUT = 32
EMB = 128
HEADS = 2


def _h0_body(x_ref, n2v_ref, Wn_ref, bn_ref, Win_ref, bin_ref, Wg_ref, bg_ref,
             o_ref):
    x = x_ref[...]
    emb_p = n2v_ref[...] @ Wn_ref[...].T + bn_ref[...]
    Win = Win_ref[...]
    raw = x @ Win[:, :D_IN].T + bin_ref[...]
    xp = raw + emb_p @ Win[:, D_IN:].T
    Wg = Wg_ref[...]
    g = jax.nn.sigmoid(x @ Wg[:, :D_IN].T + emb_p @ Wg[:, D_IN:].T + bg_ref[...])
    o_ref[...] = g * xp + (1.0 - g) * raw


def _h0(x, n2v_table, W_n2vp, b_n2vp, W_in, b_in, W_gate, b_gate):
    blk = 2000
    grid = (N // blk,)
    row_spec = pl.BlockSpec((blk, D_IN), lambda i: (i, 0))
    full = lambda shape: pl.BlockSpec(shape, lambda i: (0,) * len(shape))
    return pl.pallas_call(
        _h0_body,
        grid=grid,
        in_specs=[row_spec, row_spec, full((HID, EMB)), full((1, HID)),
                  full((HID, D_IN + HID)), full((1, HID)),
                  full((HID, D_IN + HID)), full((1, HID))],
        out_specs=pl.BlockSpec((blk, HID), lambda i: (i, 0)),
        out_shape=jax.ShapeDtypeStruct((N, HID), jnp.float32),
    )(x, n2v_table, W_n2vp, b_n2vp.reshape(1, HID), W_in,
      b_in.reshape(1, HID), W_gate, b_gate.reshape(1, HID))


def _sage(h, src, dst, Wl, bl, Wr, n, deg):
    s = jax.ops.segment_sum(h[src], dst, num_segments=n)
    deg = jax.ops.segment_sum(jnp.ones((src.shape[0],), jnp.float32), dst,
                              num_segments=n)
    mean = s / jnp.maximum(deg, 1.0)[:, None]
    return mean @ Wl.T + bl + h @ Wr.T


def _gat(h, src, dst, W, a_s, a_d, b, n):
    xw = (h @ W.T).reshape(n, HEADS, HID)
    al_s = (xw * a_s[None, :, :]).sum(-1)
    al_d = (xw * a_d[None, :, :]).sum(-1)
    e = jax.nn.leaky_relu(al_s[src] + al_d[dst], 0.2)
    m = jax.ops.segment_max(e, dst, num_segments=n)
    m = jnp.where(jnp.isfinite(m), m, 0.0)
    ex = jnp.exp(e - m[dst])
    denom = jax.ops.segment_sum(ex, dst, num_segments=n)
    coef = ex / jnp.where(denom[dst] > 0, denom[dst], 1.0)
    out = jax.ops.segment_sum(coef[:, :, None] * xw[src], dst, num_segments=n)
    return out.mean(axis=1) + b


def kernel(x, edge_index, n2v_table, W_n2vp, b_n2vp, W_in, b_in, W_gate,
           b_gate, Wl1, bl1, Wr1, Wl2, bl2, Wr2, W_gat, att_src, att_dst,
           b_gat, Wl3, bl3, Wr3):
    n = x.shape[0]
    src, dst = edge_index[0], edge_index[1]
    h0 = _h0(x, n2v_table, W_n2vp, b_n2vp, W_in, b_in, W_gate, b_gate)
    deg = jax.ops.segment_sum(jnp.ones((src.shape[0],), jnp.float32), dst,
                              num_segments=n)
    h1 = jax.nn.relu(_sage(h0, src, dst, Wl1, bl1, Wr1, n, deg))
    h2 = jax.nn.relu(_sage(h1, src, dst, Wl2, bl2, Wr2, n, deg))
    h3 = jax.nn.relu(_gat(h2, src, dst, W_gat, att_src, att_dst, b_gat, n))
    out = _sage(h3, src, dst, Wl3, bl3, Wr3, n, deg)
    return out


# speedup vs baseline: 26.3097x; 26.1901x over previous
import dataclasses
import functools

import jax
import jax.numpy as jnp
from jax import lax
from jax.experimental import pallas as pl
from jax.experimental.pallas import tpu as pltpu
from jax.experimental.pallas import tpu_sc as plsc

N = 10000
E = 320000
D_IN = 128
HID = 64
OUT = 32
EMB = 128
HEADS = 2

NC = 2
NS = 16
NW = NC * NS
EPW = E // NW
CH = 80
NCHUNK = EPW // CH
LL = 16


def _h0_body(x_ref, n2v_ref, Wn_ref, bn_ref, Win_ref, bin_ref, Wg_ref, bg_ref,
             o_ref):
    x = x_ref[...]
    emb_p = n2v_ref[...] @ Wn_ref[...].T + bn_ref[...]
    Win = Win_ref[...]
    raw = x @ Win[:, :D_IN].T + bin_ref[...]
    xp = raw + emb_p @ Win[:, D_IN:].T
    Wg = Wg_ref[...]
    g = jax.nn.sigmoid(x @ Wg[:, :D_IN].T + emb_p @ Wg[:, D_IN:].T + bg_ref[...])
    o_ref[...] = g * xp + (1.0 - g) * raw


def _h0(x, n2v_table, W_n2vp, b_n2vp, W_in, b_in, W_gate, b_gate):
    blk = 2000
    grid = (N // blk,)
    row_spec = pl.BlockSpec((blk, D_IN), lambda i: (i, 0))
    full = lambda shape: pl.BlockSpec(shape, lambda i: (0,) * len(shape))
    return pl.pallas_call(
        _h0_body,
        grid=grid,
        in_specs=[row_spec, row_spec, full((HID, EMB)), full((1, HID)),
                  full((HID, D_IN + HID)), full((1, HID)),
                  full((HID, D_IN + HID)), full((1, HID))],
        out_specs=pl.BlockSpec((blk, HID), lambda i: (i, 0)),
        out_shape=jax.ShapeDtypeStruct((N, HID), jnp.float32),
    )(x, n2v_table, W_n2vp, b_n2vp.reshape(1, HID), W_in,
      b_in.reshape(1, HID), W_gate, b_gate.reshape(1, HID))


def _stripe_copy(src_ref, dst_ref, sid, nrows):
    rp = (nrows // NS) & ~7
    tail = nrows - NS * rp
    pltpu.sync_copy(src_ref.at[pl.ds(sid * rp, rp)],
                    dst_ref.at[pl.ds(sid * rp, rp)])
    if tail:
        @pl.when(sid == 0)
        def _():
            pltpu.sync_copy(src_ref.at[pl.ds(NS * rp, tail)],
                            dst_ref.at[pl.ds(NS * rp, tail)])


def _segsum_sc(h_ext, src, dst):
    M, D = h_ext.shape
    mesh = plsc.VectorSubcoreMesh(core_axis_name="c", subcore_axis_name="s")
    zeros = jnp.zeros((M, D), jnp.float32)

    @functools.partial(
        pl.kernel,
        out_type=jax.ShapeDtypeStruct((NC, M, D), jnp.float32),
        mesh=mesh,
        compiler_params=pltpu.CompilerParams(use_tc_tiling_on_sc=False),
        scratch_types=[
            pltpu.VMEM((CH,), jnp.int32),
            pltpu.VMEM((CH,), jnp.int32),
            pltpu.VMEM((CH, D), jnp.float32),
            pltpu.VMEM_SHARED((M, D), jnp.float32),
        ],
    )
    def k(h_hbm, src_hbm, dst_hbm, z_hbm, out_hbm, src_v, dst_v, rows_v,
          acc_sh):
        cid = lax.axis_index("c")
        sid = lax.axis_index("s")
        wid = sid * NC + cid
        _stripe_copy(z_hbm, acc_sh, sid, M)
        plsc.subcore_barrier()
        base = wid * EPW

        @pl.loop(0, NCHUNK)
        def _(i):
            off = base + i * CH
            pltpu.sync_copy(src_hbm.at[pl.ds(off, CH)], src_v)
            pltpu.sync_copy(dst_hbm.at[pl.ds(off, CH)], dst_v)
            pltpu.sync_copy(h_hbm.at[src_v], rows_v)
            pltpu.sync_copy(rows_v, acc_sh.at[dst_v], add=True)

        plsc.subcore_barrier()
        _stripe_copy(acc_sh, out_hbm.at[cid], sid, M)

    return k(h_ext, src, dst, zeros)


def _segsum_cond_sc(T, als, ald, src, dst):
    M, D = T.shape
    mesh = plsc.VectorSubcoreMesh(core_axis_name="c", subcore_axis_name="s")
    zeros = jnp.zeros((M, D), jnp.float32)
    cp = pltpu.CompilerParams(use_tc_tiling_on_sc=False)
    if "needs_layout_passes" in pltpu.CompilerParams.__dataclass_fields__:
        cp = dataclasses.replace(cp, needs_layout_passes=False)

    @functools.partial(
        pl.kernel,
        out_type=jax.ShapeDtypeStruct((NC, M, D), jnp.float32),
        mesh=mesh,
        compiler_params=cp,
        scratch_types=[
            pltpu.VMEM((CH,), jnp.int32),
            pltpu.VMEM((CH,), jnp.int32),
            pltpu.VMEM((CH,), jnp.int32),
            pltpu.VMEM((CH,), jnp.int32),
            pltpu.VMEM((CH, D), jnp.float32),
            pltpu.VMEM((N,), jnp.float32),
            pltpu.VMEM((N,), jnp.float32),
            pltpu.VMEM_SHARED((M, D), jnp.float32),
        ],
    )
    def k(t_hbm, als_hbm, ald_hbm, src_hbm, dst_hbm, z_hbm, out_hbm,
          src_v, dst_v, gidx_v, sidx_v, rows_v, als_v, ald_v, acc_sh):
        cid = lax.axis_index("c")
        sid = lax.axis_index("s")
        wid = sid * NC + cid
        pltpu.sync_copy(als_hbm, als_v)
        pltpu.sync_copy(ald_hbm, ald_v)
        _stripe_copy(z_hbm, acc_sh, sid, M)
        plsc.subcore_barrier()
        base = wid * EPW

        @pl.loop(0, NCHUNK)
        def _(i):
            off = base + i * CH
            pltpu.sync_copy(src_hbm.at[pl.ds(off, CH)], src_v)
            pltpu.sync_copy(dst_hbm.at[pl.ds(off, CH)], dst_v)
            for kk in range(CH // LL):
                s16 = src_v[pl.ds(kk * LL, LL)]
                d16 = dst_v[pl.ds(kk * LL, LL)]
                a = plsc.load_gather(als_v, [s16])
                b = plsc.load_gather(ald_v, [d16])
                z = a + b
                offv = jnp.where(z > 0.0,
                                 jnp.full((LL,), 0, jnp.int32),
                                 jnp.full((LL,), N, jnp.int32))
                gidx_v[pl.ds(kk * LL, LL)] = s16 + offv
                sidx_v[pl.ds(kk * LL, LL)] = d16 + offv
            pltpu.sync_copy(t_hbm.at[gidx_v], rows_v)
            pltpu.sync_copy(rows_v, acc_sh.at[sidx_v], add=True)

        plsc.subcore_barrier()
        _stripe_copy(acc_sh, out_hbm.at[cid], sid, M)

    return k(T, als, ald, src, dst, zeros)


def _sage_post(p, h, Wl, bl, Wr, deg_inv):
    s = p[0, :, :HID] + p[1, :, :HID]
    if deg_inv is None:
        deg = p[0, :, HID] + p[1, :, HID]
        deg_inv = 1.0 / jnp.maximum(deg, 1.0)
    mean = s * deg_inv[:, None]
    return mean @ Wl.T + bl + h @ Wr.T, deg_inv


def _gat_head(xw_h, als, ald, src, dst):
    n = xw_h.shape[0]
    asm = jnp.max(als)
    adm = jnp.max(ald)
    bb = asm + adm
    mb = jnp.maximum(bb, 0.2 * bb)
    fpos = jnp.exp(als - asm)
    fneg = jnp.exp(0.2 * (als - asm))
    pad = jnp.zeros((n, 15), jnp.float32)
    Tp = jnp.concatenate([fpos[:, None] * xw_h, fpos[:, None], pad], axis=1)
    Tn = jnp.concatenate([fneg[:, None] * xw_h, fneg[:, None], pad], axis=1)
    T = jnp.concatenate([Tp, Tn], axis=0)
    p = _segsum_cond_sc(T, als, ald, src, dst)
    S = p[0] + p[1]
    gpos = jnp.exp(ald - adm + bb - mb)
    gneg = jnp.exp(0.2 * (ald - adm) + 0.2 * bb - mb)
    num = gpos[:, None] * S[:n, :HID] + gneg[:, None] * S[n:, :HID]
    den = gpos * S[:n, HID] + gneg * S[n:, HID]
    return num, den


def kernel(x, edge_index, n2v_table, W_n2vp, b_n2vp, W_in, b_in, W_gate,
           b_gate, Wl1, bl1, Wr1, Wl2, bl2, Wr2, W_gat, att_src, att_dst,
           b_gat, Wl3, bl3, Wr3):
    n = x.shape[0]
    src = edge_index[0].astype(jnp.int32)
    dst = edge_index[1].astype(jnp.int32)
    h0 = _h0(x, n2v_table, W_n2vp, b_n2vp, W_in, b_in, W_gate, b_gate)
    h0e = jnp.concatenate(
        [h0, jnp.ones((n, 1), jnp.float32), jnp.zeros((n, 15), jnp.float32)],
        axis=1)
    h1, deg_inv = _sage_post(_segsum_sc(h0e, src, dst), h0, Wl1, bl1, Wr1,
                             None)
    h1 = jax.nn.relu(h1)
    h2, _ = _sage_post(_segsum_sc(h1, src, dst), h1, Wl2, bl2, Wr2, deg_inv)
    h2 = jax.nn.relu(h2)

    xw = h2 @ W_gat.T
    outs = []
    for h in range(HEADS):
        xw_h = xw[:, h * HID:(h + 1) * HID]
        als = xw_h @ att_src[h]
        ald = xw_h @ att_dst[h]
        num, den = _gat_head(xw_h, als, ald, src, dst)
        outs.append(num / jnp.where(den > 0, den, 1.0)[:, None])
    h3 = jax.nn.relu((outs[0] + outs[1]) * 0.5 + b_gat)

    p = _segsum_sc(h3, src, dst)
    s = p[0] + p[1]
    mean = s * deg_inv[:, None]
    return mean @ Wl3.T + bl3 + h3 @ Wr3.T
